# baseline (speedup 1.0000x reference)
import jax
import jax.numpy as jnp
from jax.experimental import pallas as pl
from jax.experimental.pallas import tpu as pltpu

_N = 10000
_F = 512
_BM = 400
_BX = 2000
_NX = _N // _BX


def _fused_kernel(x_ref, w_ref, adj1_ref, adj2_ref, b_ref, a_ref, out_ref,
                  h_ref):
    i = pl.program_id(0)

    @pl.when(i < _NX)
    def _():
        h = jax.lax.dot_general(
            x_ref[:], w_ref[:], (((1,), (1,)), ((), ())),
            preferred_element_type=jnp.float32)
        h_ref[pl.ds(i * _BX, _BX), :] = h.astype(jnp.bfloat16)

    @pl.when(i >= _NX)
    def _():
        o1 = jax.lax.dot_general(
            adj1_ref[:].astype(jnp.bfloat16), h_ref[:],
            (((1,), (0,)), ((), ())),
            preferred_element_type=jnp.float32)
        o2 = jax.lax.dot_general(
            adj2_ref[:].astype(jnp.bfloat16), h_ref[:],
            (((1,), (0,)), ((), ())),
            preferred_element_type=jnp.float32)
        o = jnp.concatenate([o1, o2], axis=0) + b_ref[:]
        out_ref[:] = jnp.where(o >= 0, o, a_ref[0, 0] * o)


def kernel(x, adj, W, b, prelu_a):
    grid = _N // _BM + _NX
    out = pl.pallas_call(
        _fused_kernel,
        grid=(grid,),
        in_specs=[
            pl.BlockSpec((_BX, _F), lambda i: (jnp.minimum(i, _NX - 1), 0)),
            pl.BlockSpec((_F, _F), lambda i: (0, 0)),
            pl.BlockSpec((_BM // 2, _N),
                         lambda i: (2 * jnp.maximum(i - _NX, 0), 0)),
            pl.BlockSpec((_BM // 2, _N),
                         lambda i: (2 * jnp.maximum(i - _NX, 0) + 1, 0)),
            pl.BlockSpec((1, _F), lambda i: (0, 0)),
            pl.BlockSpec((1, 1), lambda i: (0, 0)),
        ],
        out_specs=pl.BlockSpec((_BM, _F), lambda i: (jnp.maximum(i - _NX, 0), 0)),
        out_shape=jax.ShapeDtypeStruct((_N, _F), jnp.float32),
        scratch_shapes=[pltpu.VMEM((_N, _F), jnp.bfloat16)],
        compiler_params=pltpu.CompilerParams(
            dimension_semantics=("arbitrary",)),
    )(x, W, adj, adj, b.reshape(1, _F), prelu_a.reshape(1, 1))
    return out


# speedup vs baseline: 1.0306x; 1.0306x over previous
import jax
import jax.numpy as jnp
from jax.experimental import pallas as pl
from jax.experimental.pallas import tpu as pltpu

_N = 10000
_F = 512
_BM = 400
_BX = 2000
_NX = _N // _BX


def _fused_kernel(x_ref, w_ref, adj_ref, b_ref, a_ref, out_ref, h_ref):
    i = pl.program_id(0)

    @pl.when(i < _NX)
    def _():
        h = jax.lax.dot_general(
            x_ref[:], w_ref[:], (((1,), (1,)), ((), ())),
            preferred_element_type=jnp.float32)
        h_ref[pl.ds(i * _BX, _BX), :] = h.astype(jnp.bfloat16)

    @pl.when(i >= _NX)
    def _():
        o = jax.lax.dot_general(
            adj_ref[:].astype(jnp.bfloat16), h_ref[:],
            (((1,), (0,)), ((), ())),
            preferred_element_type=jnp.float32)
        o = o + b_ref[:]
        out_ref[:] = jnp.where(o >= 0, o, a_ref[0, 0] * o)


def kernel(x, adj, W, b, prelu_a):
    grid = _N // _BM + _NX
    out = pl.pallas_call(
        _fused_kernel,
        grid=(grid,),
        in_specs=[
            pl.BlockSpec((_BX, _F), lambda i: (jnp.minimum(i, _NX - 1), 0)),
            pl.BlockSpec((_F, _F), lambda i: (0, 0)),
            pl.BlockSpec((_BM, _N), lambda i: (jnp.maximum(i - _NX, 0), 0)),
            pl.BlockSpec((1, _F), lambda i: (0, 0)),
            pl.BlockSpec((1, 1), lambda i: (0, 0)),
        ],
        out_specs=pl.BlockSpec((_BM, _F), lambda i: (jnp.maximum(i - _NX, 0), 0)),
        out_shape=jax.ShapeDtypeStruct((_N, _F), jnp.float32),
        scratch_shapes=[pltpu.VMEM((_N, _F), jnp.bfloat16)],
        compiler_params=pltpu.CompilerParams(
            dimension_semantics=("arbitrary",)),
    )(x, W, adj, b.reshape(1, _F), prelu_a.reshape(1, 1))
    return out
